# Initial kernel scaffold; baseline (speedup 1.0000x reference)
import jax
import jax.numpy as jnp
from jax.experimental import pallas as pl


def kernel(r_x, rn_x, p_x, pn_x, r_e, p_e, r_ei, p_ei, r_gid, p_gid, gin, ginn, Wq, Wk, Wp, bp):
    raise NotImplementedError("write your pallas kernel here")


# speedup vs baseline: 2.2970x; 2.2970x over previous
import functools
import math

import jax
import jax.numpy as jnp
from jax import lax
from jax.experimental import pallas as pl
from jax.experimental.pallas import tpu as pltpu
from jax.experimental.pallas import tpu_sc as plsc

R, P, B = 3, 2, 512
N, E = 32768, 65536
D_IN, D_E, H, OUT = 155, 9, 64, 4
DEPTH = 3
NC, NS = 2, 16
HH = H // NC
G = 2 * (R + P)

_SC_PARAMS = pltpu.CompilerParams(use_tc_tiling_on_sc=False)


def _mesh():
    return plsc.VectorSubcoreMesh(core_axis_name="c", subcore_axis_name="s",
                                  num_cores=NC, num_subcores=NS)


def _node_proj(X, W, b):
    S, n, D = X.shape
    bn = 2048

    def body(x_ref, w_ref, b_ref, o_ref):
        h = jnp.dot(x_ref[0], w_ref[...],
                    preferred_element_type=jnp.float32, precision=lax.Precision.HIGHEST) + b_ref[...]
        o_ref[0, 0] = h[:, :HH]
        o_ref[1, 0] = h[:, HH:]

    return pl.pallas_call(
        body,
        grid=(S, n // bn),
        in_specs=[
            pl.BlockSpec((1, bn, D), lambda g, i: (g, i, 0)),
            pl.BlockSpec((D, H), lambda g, i: (0, 0)),
            pl.BlockSpec((1, H), lambda g, i: (0, 0)),
        ],
        out_specs=pl.BlockSpec((NC, 1, bn, HH), lambda g, i: (0, g, i, 0)),
        out_shape=jax.ShapeDtypeStruct((NC, S, n, HH), jnp.float32),
    )(X, W, b.reshape(1, H))


def _edge_proj(EA, Ws, bs):
    be = 2048

    def body(e_ref, w_ref, b_ref, o_ref):
        h = jnp.dot(e_ref[0], w_ref[0],
                    preferred_element_type=jnp.float32, precision=lax.Precision.HIGHEST) + b_ref[0]
        o_ref[0, 0, 0] = h[:, :HH]
        o_ref[1, 0, 0] = h[:, HH:]

    return pl.pallas_call(
        body,
        grid=(2, 5, E // be),
        in_specs=[
            pl.BlockSpec((1, be, D_E), lambda p, g, i: (g, i, 0)),
            pl.BlockSpec((1, D_E, H), lambda p, g, i: (p, 0, 0)),
            pl.BlockSpec((1, 1, H), lambda p, g, i: (p, 0, 0)),
        ],
        out_specs=pl.BlockSpec((NC, 1, 1, be, HH),
                               lambda p, g, i: (0, p, g, i, 0)),
        out_shape=jax.ShapeDtypeStruct((NC, 2, 5, E, HH), jnp.float32),
    )(EA, Ws, bs.reshape(2, 1, H))


def _update(Hp, Ap, W1, b1, W2, b2):
    bn = 2048

    def body(h_ref, a_ref, w1_ref, b1_ref, w2_ref, b2_ref, o_ref):
        h = jnp.concatenate([h_ref[0, 0], h_ref[1, 0]], axis=-1)
        a = jnp.concatenate([a_ref[0, 0], a_ref[1, 0]], axis=-1)
        z = h + a
        z = jnp.maximum(jnp.dot(z, w1_ref[...],
                                preferred_element_type=jnp.float32, precision=lax.Precision.HIGHEST)
                        + b1_ref[...], 0.0)
        z = jnp.dot(z, w2_ref[...],
                    preferred_element_type=jnp.float32, precision=lax.Precision.HIGHEST) + b2_ref[...]
        z = jnp.maximum(z, 0.0)
        o_ref[0, 0] = z[:, :HH]
        o_ref[1, 0] = z[:, HH:]

    return pl.pallas_call(
        body,
        grid=(5, N // bn),
        in_specs=[
            pl.BlockSpec((NC, 1, bn, HH), lambda g, i: (0, g, i, 0)),
            pl.BlockSpec((NC, 1, bn, HH), lambda g, i: (0, g, i, 0)),
            pl.BlockSpec((H, H), lambda g, i: (0, 0)),
            pl.BlockSpec((1, H), lambda g, i: (0, 0)),
            pl.BlockSpec((H, H), lambda g, i: (0, 0)),
            pl.BlockSpec((1, H), lambda g, i: (0, 0)),
        ],
        out_specs=pl.BlockSpec((NC, 1, bn, HH), lambda g, i: (0, g, i, 0)),
        out_shape=jax.ShapeDtypeStruct((NC, 5, N, HH), jnp.float32),
    )(Hp, Ap, W1, b1.reshape(1, H), W2, b2.reshape(1, H))


def _attn(sums, cnt, Wq, Wk, Wp, bp):
    sc = 1.0 / math.sqrt(H)

    def body(s_ref, c_ref, wq_ref, wk_ref, wp_ref, bp_ref,
             o_out, o_ar, o_ap):
        feats = []
        for g in range(G):
            f = jnp.concatenate([s_ref[0, g], s_ref[1, g]], axis=-1)
            gsel = 0 if (g % 5) < R else 1
            cc = c_ref[gsel][:, :1]
            feats.append(f / jnp.maximum(cc, 1.0))
        r = feats[0:3]
        p = feats[3:5]
        rn = feats[5:8]
        pn = feats[8:10]
        r_rows = [r[0], r[1], r[2], r[0] + r[1], r[0] + r[2], r[1] + r[2]]
        rn_rows = [rn[0], rn[1], rn[2], rn[0] + rn[1], rn[0] + rn[2],
                   rn[1] + rn[2]]
        p_rows = [p[0], p[1], p[0] + p[1]]
        pn_rows = [pn[0], pn[1], pn[0] + pn[1]]

        def dotw(x, w):
            return jnp.dot(x, w, preferred_element_type=jnp.float32, precision=lax.Precision.HIGHEST)

        def att(q_list, k_list):
            acc = None
            for q in q_list:
                sij = jnp.concatenate(
                    [jnp.sum(q * k, axis=-1, keepdims=True) for k in k_list],
                    axis=-1) * sc
                m = jnp.max(sij, axis=-1, keepdims=True)
                e = jnp.exp(sij - m)
                a = e / jnp.sum(e, axis=-1, keepdims=True)
                acc = a if acc is None else acc + a
            return acc / float(len(q_list))

        qp = [dotw(x, wq_ref[...]) for x in p_rows]
        kr = [dotw(x, wk_ref[...]) for x in r_rows]
        att_reactant = att(qp, kr)
        qr = [dotw(x, wq_ref[...]) for x in r_rows]
        kp = [dotw(x, wk_ref[...]) for x in p_rows]
        att_product = att(qr, kp)

        reactant = sum(att_reactant[:, k:k + 1] * (r_rows[k] + rn_rows[k])
                       for k in range(6))
        product = sum(att_product[:, k:k + 1] * (p_rows[k] + pn_rows[k])
                      for k in range(3))
        reaction = reactant - product
        o_out[...] = jnp.dot(reaction, wp_ref[...],
                             preferred_element_type=jnp.float32, precision=lax.Precision.HIGHEST) + bp_ref[...]
        o_ar[...] = att_reactant
        o_ap[...] = att_product

    return pl.pallas_call(
        body,
        out_shape=(
            jax.ShapeDtypeStruct((B, OUT), jnp.float32),
            jax.ShapeDtypeStruct((B, 6), jnp.float32),
            jax.ShapeDtypeStruct((B, 3), jnp.float32),
        ),
    )(sums, cnt, Wq, Wk, Wp, bp.reshape(1, OUT))


def _msg(Hg, Hn, Eall, ei128):
    ec = 512
    nchunks = E // NS // ec

    @functools.partial(
        pl.kernel, mesh=_mesh(), compiler_params=_SC_PARAMS,
        out_type=(jax.ShapeDtypeStruct((NC, 5, N, HH), jnp.float32),
                  jax.ShapeDtypeStruct((NC, 5, N, HH), jnp.float32)),
        scratch_types=[
            pltpu.VMEM((ec // 128, 128), jnp.int32),
            pltpu.VMEM((ec // 128, 128), jnp.int32),
            pltpu.VMEM((ec, HH), jnp.float32),
            pltpu.VMEM((ec, HH), jnp.float32),
            pltpu.VMEM((ec, HH), jnp.float32),
            pltpu.VMEM_SHARED((N, HH), jnp.float32),
            pltpu.SemaphoreType.DMA,
        ],
    )
    def msg(hg, hn, eall, ei128ref, agg_g, agg_n,
            sidx_v, didx_v, rows_v, m_v, z_v, agg_sp, sem):
        c = lax.axis_index("c")
        s = lax.axis_index("s")

        @plsc.parallel_loop(0, ec, unroll=8)
        def _zz(i):
            z_v[i, pl.ds(0, 16)] = jnp.zeros((16,), jnp.float32)
            z_v[i, pl.ds(16, 16)] = jnp.zeros((16,), jnp.float32)

        for g in range(G):
            pg, s5 = (0, g) if g < 5 else (1, g - 5)
            href = hg if g < 5 else hn
            aref = agg_g if g < 5 else agg_n
            for q in range(N // NS // ec):
                pltpu.sync_copy(z_v,
                                agg_sp.at[pl.ds(s * (N // NS) + q * ec, ec)])
            plsc.subcore_barrier()

            def chunk(ck, _):
                eb = s * (E // NS) + ck * ec
                pltpu.sync_copy(
                    ei128ref.at[s5, 0, pl.ds(eb // 128, ec // 128)], sidx_v)
                pltpu.sync_copy(
                    ei128ref.at[s5, 1, pl.ds(eb // 128, ec // 128)], didx_v)
                gcps = [
                    pltpu.async_copy(href.at[c, s5].at[sidx_v.at[j]],
                                     rows_v.at[pl.ds(j * 128, 128)], sem)
                    for j in range(ec // 128)
                ]
                pltpu.sync_copy(eall.at[c, pg, s5, pl.ds(eb, ec)], m_v)
                for gcp in gcps:
                    gcp.wait()

                @plsc.parallel_loop(0, ec, unroll=8)
                def _cm(i):
                    a = m_v[i, pl.ds(0, 16)] + rows_v[i, pl.ds(0, 16)]
                    m_v[i, pl.ds(0, 16)] = jnp.maximum(a, 0.0)
                    b2 = m_v[i, pl.ds(16, 16)] + rows_v[i, pl.ds(16, 16)]
                    m_v[i, pl.ds(16, 16)] = jnp.maximum(b2, 0.0)

                for j in range(ec // 128):
                    pltpu.sync_copy(m_v.at[pl.ds(j * 128, 128)],
                                    agg_sp.at[didx_v.at[j]], add=True)
                return 0

            lax.fori_loop(0, nchunks, chunk, 0)
            plsc.subcore_barrier()
            pltpu.sync_copy(
                agg_sp.at[pl.ds(s * (N // NS), N // NS)],
                aref.at[c, s5, pl.ds(s * (N // NS), N // NS)])

    return msg(Hg, Hn, Eall, ei128)


def _readout(Hg, Hn, gid128):
    @functools.partial(
        pl.kernel, mesh=_mesh(), compiler_params=_SC_PARAMS,
        out_type=(jax.ShapeDtypeStruct((NC, G, B, HH), jnp.float32),
                  jax.ShapeDtypeStruct((NC, B, HH), jnp.float32)),
        scratch_types=[
            pltpu.VMEM((4, 128), jnp.int32),
            pltpu.VMEM((512, HH), jnp.float32),
            pltpu.VMEM((128, HH), jnp.float32),
            pltpu.VMEM((B // NS, HH), jnp.float32),
            pltpu.VMEM_SHARED((B, HH), jnp.float32),
            pltpu.VMEM_SHARED((B, HH), jnp.float32),
        ],
    )
    def rd(hg, hn, gidref, sums, cnt,
           didx_v, m_v, ones_v, z_v, sums_sp, cnt_sp):
        c = lax.axis_index("c")
        s = lax.axis_index("s")

        @plsc.parallel_loop(0, 128, unroll=8)
        def _io(i):
            ones_v[i, pl.ds(0, 16)] = jnp.ones((16,), jnp.float32)
            ones_v[i, pl.ds(16, 16)] = jnp.ones((16,), jnp.float32)

        @plsc.parallel_loop(0, B // NS, unroll=8)
        def _iz(i):
            z_v[i, pl.ds(0, 16)] = jnp.zeros((16,), jnp.float32)
            z_v[i, pl.ds(16, 16)] = jnp.zeros((16,), jnp.float32)

        pltpu.sync_copy(z_v, cnt_sp.at[pl.ds(s * (B // NS), B // NS)])
        plsc.subcore_barrier()

        def cchunk(ck, _):
            rb = s * 16 + ck * 4
            pltpu.sync_copy(gidref.at[c, pl.ds(rb, 4)], didx_v)
            for j in range(4):
                pltpu.sync_copy(ones_v, cnt_sp.at[didx_v.at[j]], add=True)
            return 0

        lax.fori_loop(0, 4, cchunk, 0)
        plsc.subcore_barrier()
        pltpu.sync_copy(cnt_sp.at[pl.ds(s * (B // NS), B // NS)],
                        cnt.at[c, pl.ds(s * (B // NS), B // NS)])

        for g in range(G):
            s5 = g % 5
            href = hg if g < 5 else hn
            gsel = 0 if s5 < R else 1
            pltpu.sync_copy(z_v, sums_sp.at[pl.ds(s * (B // NS), B // NS)])
            plsc.subcore_barrier()

            def schunk(ck, _):
                nb = s * (N // NS) + ck * 512
                rb = s * 16 + ck * 4
                pltpu.sync_copy(gidref.at[gsel, pl.ds(rb, 4)], didx_v)
                pltpu.sync_copy(href.at[c, s5, pl.ds(nb, 512)], m_v)
                for j in range(4):
                    pltpu.sync_copy(m_v.at[pl.ds(j * 128, 128)],
                                    sums_sp.at[didx_v.at[j]], add=True)
                return 0

            lax.fori_loop(0, 4, schunk, 0)
            plsc.subcore_barrier()
            pltpu.sync_copy(sums_sp.at[pl.ds(s * (B // NS), B // NS)],
                            sums.at[c, g, pl.ds(s * (B // NS), B // NS)])

    return rd(Hg, Hn, gid128)


def kernel(r_x, rn_x, p_x, pn_x, r_e, p_e, r_ei, p_ei, r_gid, p_gid,
           gin, ginn, Wq, Wk, Wp, bp):
    Xg = jnp.concatenate([r_x, p_x], axis=0)
    Xn = jnp.concatenate([rn_x, pn_x], axis=0)
    EA = jnp.concatenate([r_e, p_e], axis=0)
    ei128 = jnp.concatenate([r_ei, p_ei], axis=0).reshape(5, 2, E // 128, 128)
    gid128 = jnp.stack([r_gid, p_gid], axis=0).reshape(2, N // 128, 128)

    Hg = _node_proj(Xg, gin['Wn'], gin['bn'])
    Hn = _node_proj(Xn, ginn['Wn'], ginn['bn'])
    Eall = _edge_proj(EA,
                      jnp.stack([gin['We'], ginn['We']], axis=0),
                      jnp.stack([gin['be'], ginn['be']], axis=0))

    for l in range(DEPTH):
        Ag, An = _msg(Hg, Hn, Eall, ei128)
        Hg = _update(Hg, Ag, gin['l%d_W1' % l], gin['l%d_b1' % l],
                     gin['l%d_W2' % l], gin['l%d_b2' % l])
        Hn = _update(Hn, An, ginn['l%d_W1' % l], ginn['l%d_b1' % l],
                     ginn['l%d_W2' % l], ginn['l%d_b2' % l])

    sums, cnt = _readout(Hg, Hn, gid128)
    return _attn(sums, cnt, Wq, Wk, Wp, bp)


# speedup vs baseline: 4.2752x; 1.8612x over previous
import functools
import math

import jax
import jax.numpy as jnp
from jax import lax
from jax.experimental import pallas as pl
from jax.experimental.pallas import tpu as pltpu
from jax.experimental.pallas import tpu_sc as plsc

R, P, B = 3, 2, 512
N, E = 32768, 65536
D_IN, D_E, H, OUT = 155, 9, 64, 4
DEPTH = 3
NC, NS = 2, 16
HH = H // NC
G = 2 * (R + P)

_SC_PARAMS = pltpu.CompilerParams(use_tc_tiling_on_sc=False)


def _mesh():
    return plsc.VectorSubcoreMesh(core_axis_name="c", subcore_axis_name="s",
                                  num_cores=NC, num_subcores=NS)


def _expand_w1(W1):
    A = jnp.zeros((128, 256), W1.dtype)
    Bm = jnp.zeros((128, 256), W1.dtype)
    for j in range(4):
        A = A.at[32 * j:32 * j + 32, 64 * j:64 * j + 64].set(W1[:HH])
        Bm = Bm.at[32 * j:32 * j + 32, 64 * j:64 * j + 64].set(W1[HH:])
    return A, Bm


def _expand_w2(W2):
    C = jnp.zeros((256, 256), W2.dtype)
    for j in range(4):
        C = C.at[64 * j:64 * j + 64, 32 * j:32 * j + 32].set(W2[:, :HH])
        C = C.at[64 * j:64 * j + 64,
                 128 + 32 * j:128 + 32 * j + 32].set(W2[:, HH:])
    return C


def _expand_proj(W):
    D = W.shape[0]
    W4 = jnp.zeros((4 * D, 256), W.dtype)
    for j in range(4):
        W4 = W4.at[D * j:D * j + D, 32 * j:32 * j + 32].set(W[:, :HH])
        W4 = W4.at[D * j:D * j + D,
                   128 + 32 * j:128 + 32 * j + 32].set(W[:, HH:])
    return W4


def _pack_bias(b):
    return jnp.concatenate([jnp.tile(b[:HH], 4), jnp.tile(b[HH:], 4)])


def _dot(x, w):
    return jnp.dot(x, w, preferred_element_type=jnp.float32,
                   precision=lax.Precision.HIGHEST)


def _node_proj(X4, W4, b4):
    S, n4, D4 = X4.shape
    bn = 2048

    def body(x_ref, w_ref, b_ref, o_ref):
        y = _dot(x_ref[0], w_ref[...]) + b_ref[...]
        o_ref[0, 0] = y[:, :128]
        o_ref[1, 0] = y[:, 128:]

    return pl.pallas_call(
        body,
        grid=(S, n4 // bn),
        in_specs=[
            pl.BlockSpec((1, bn, D4), lambda g, i: (g, i, 0)),
            pl.BlockSpec((D4, 256), lambda g, i: (0, 0)),
            pl.BlockSpec((1, 256), lambda g, i: (0, 0)),
        ],
        out_specs=pl.BlockSpec((NC, 1, bn, 128), lambda g, i: (0, g, i, 0)),
        out_shape=jax.ShapeDtypeStruct((NC, S, n4, 128), jnp.float32),
    )(X4, W4, b4.reshape(1, 256))


def _edge_proj(EA4, W4s, b4s):
    be = 2048
    e4 = E // 4

    def body(e_ref, w_ref, b_ref, o_ref):
        y = _dot(e_ref[0], w_ref[0]) + b_ref[0]
        o_ref[0, 0, 0] = y[:, :128]
        o_ref[1, 0, 0] = y[:, 128:]

    return pl.pallas_call(
        body,
        grid=(2, 5, e4 // be),
        in_specs=[
            pl.BlockSpec((1, be, 4 * D_E), lambda p, g, i: (g, i, 0)),
            pl.BlockSpec((1, 4 * D_E, 256), lambda p, g, i: (p, 0, 0)),
            pl.BlockSpec((1, 1, 256), lambda p, g, i: (p, 0, 0)),
        ],
        out_specs=pl.BlockSpec((NC, 1, 1, be, 128),
                               lambda p, g, i: (0, p, g, i, 0)),
        out_shape=jax.ShapeDtypeStruct((NC, 2, 5, e4, 128), jnp.float32),
    )(EA4, W4s, b4s.reshape(2, 1, 256))


def _update(Hp4, Ap4, A1, B1, b1t, C2, b2p):
    bn = 2048
    n4 = N // 4

    def body(h_ref, a_ref, A_ref, B_ref, b1_ref, C_ref, b2_ref, o_ref):
        s0 = h_ref[0, 0] + a_ref[0, 0]
        s1 = h_ref[1, 0] + a_ref[1, 0]
        z1 = _dot(s0, A_ref[...]) + _dot(s1, B_ref[...]) + b1_ref[...]
        z1 = jnp.maximum(z1, 0.0)
        z2 = _dot(z1, C_ref[...]) + b2_ref[...]
        z2 = jnp.maximum(z2, 0.0)
        o_ref[0, 0] = z2[:, :128]
        o_ref[1, 0] = z2[:, 128:]

    return pl.pallas_call(
        body,
        grid=(5, n4 // bn),
        in_specs=[
            pl.BlockSpec((NC, 1, bn, 128), lambda g, i: (0, g, i, 0)),
            pl.BlockSpec((NC, 1, bn, 128), lambda g, i: (0, g, i, 0)),
            pl.BlockSpec((128, 256), lambda g, i: (0, 0)),
            pl.BlockSpec((128, 256), lambda g, i: (0, 0)),
            pl.BlockSpec((1, 256), lambda g, i: (0, 0)),
            pl.BlockSpec((256, 256), lambda g, i: (0, 0)),
            pl.BlockSpec((1, 256), lambda g, i: (0, 0)),
        ],
        out_specs=pl.BlockSpec((NC, 1, bn, 128), lambda g, i: (0, g, i, 0)),
        out_shape=jax.ShapeDtypeStruct((NC, 5, n4, 128), jnp.float32),
    )(Hp4, Ap4, A1, B1, b1t.reshape(1, 256), C2, b2p.reshape(1, 256))


def _attn(sums, cnt, Wq, Wk, Wp, bp):
    sc = 1.0 / math.sqrt(H)

    def body(s_ref, c_ref, wq_ref, wk_ref, wp_ref, bp_ref,
             o_out, o_ar, o_ap):
        feats = []
        for g in range(G):
            f = jnp.concatenate([s_ref[0, g], s_ref[1, g]], axis=-1)
            gsel = 0 if (g % 5) < R else 1
            cc = c_ref[gsel][:, :1]
            feats.append(f / jnp.maximum(cc, 1.0))
        r = feats[0:3]
        p = feats[3:5]
        rn = feats[5:8]
        pn = feats[8:10]
        r_rows = [r[0], r[1], r[2], r[0] + r[1], r[0] + r[2], r[1] + r[2]]
        rn_rows = [rn[0], rn[1], rn[2], rn[0] + rn[1], rn[0] + rn[2],
                   rn[1] + rn[2]]
        p_rows = [p[0], p[1], p[0] + p[1]]
        pn_rows = [pn[0], pn[1], pn[0] + pn[1]]

        def dotw(x, w):
            return jnp.dot(x, w, preferred_element_type=jnp.float32, precision=lax.Precision.HIGHEST)

        def att(q_list, k_list):
            acc = None
            for q in q_list:
                sij = jnp.concatenate(
                    [jnp.sum(q * k, axis=-1, keepdims=True) for k in k_list],
                    axis=-1) * sc
                m = jnp.max(sij, axis=-1, keepdims=True)
                e = jnp.exp(sij - m)
                a = e / jnp.sum(e, axis=-1, keepdims=True)
                acc = a if acc is None else acc + a
            return acc / float(len(q_list))

        qp = [dotw(x, wq_ref[...]) for x in p_rows]
        kr = [dotw(x, wk_ref[...]) for x in r_rows]
        att_reactant = att(qp, kr)
        qr = [dotw(x, wq_ref[...]) for x in r_rows]
        kp = [dotw(x, wk_ref[...]) for x in p_rows]
        att_product = att(qr, kp)

        reactant = sum(att_reactant[:, k:k + 1] * (r_rows[k] + rn_rows[k])
                       for k in range(6))
        product = sum(att_product[:, k:k + 1] * (p_rows[k] + pn_rows[k])
                      for k in range(3))
        reaction = reactant - product
        o_out[...] = jnp.dot(reaction, wp_ref[...],
                             preferred_element_type=jnp.float32, precision=lax.Precision.HIGHEST) + bp_ref[...]
        o_ar[...] = att_reactant
        o_ap[...] = att_product

    return pl.pallas_call(
        body,
        out_shape=(
            jax.ShapeDtypeStruct((B, OUT), jnp.float32),
            jax.ShapeDtypeStruct((B, 6), jnp.float32),
            jax.ShapeDtypeStruct((B, 3), jnp.float32),
        ),
    )(sums, cnt, Wq, Wk, Wp, bp.reshape(1, OUT))


def _msg(Hg, Hn, Eall, ei128):
    ec = 512
    nchunks = E // NS // ec

    @functools.partial(
        pl.kernel, mesh=_mesh(), compiler_params=_SC_PARAMS,
        out_type=(jax.ShapeDtypeStruct((NC, 5, N, HH), jnp.float32),
                  jax.ShapeDtypeStruct((NC, 5, N, HH), jnp.float32)),
        scratch_types=[
            pltpu.VMEM((ec // 128, 128), jnp.int32),
            pltpu.VMEM((ec // 128, 128), jnp.int32),
            pltpu.VMEM((ec, HH), jnp.float32),
            pltpu.VMEM((ec, HH), jnp.float32),
            pltpu.VMEM((ec, HH), jnp.float32),
            pltpu.VMEM_SHARED((N, HH), jnp.float32),
            pltpu.SemaphoreType.DMA,
        ],
    )
    def msg(hg, hn, eall, ei128ref, agg_g, agg_n,
            sidx_v, didx_v, rows_v, m_v, z_v, agg_sp, sem):
        c = lax.axis_index("c")
        s = lax.axis_index("s")

        @plsc.parallel_loop(0, ec, unroll=8)
        def _zz(i):
            z_v[i, pl.ds(0, 16)] = jnp.zeros((16,), jnp.float32)
            z_v[i, pl.ds(16, 16)] = jnp.zeros((16,), jnp.float32)

        for g in range(G):
            pg, s5 = (0, g) if g < 5 else (1, g - 5)
            href = hg if g < 5 else hn
            aref = agg_g if g < 5 else agg_n
            for q in range(N // NS // ec):
                pltpu.sync_copy(z_v,
                                agg_sp.at[pl.ds(s * (N // NS) + q * ec, ec)])
            plsc.subcore_barrier()

            def chunk(ck, _):
                eb = s * (E // NS) + ck * ec
                pltpu.sync_copy(
                    ei128ref.at[s5, 0, pl.ds(eb // 128, ec // 128)], sidx_v)
                pltpu.sync_copy(
                    ei128ref.at[s5, 1, pl.ds(eb // 128, ec // 128)], didx_v)
                gcps = [
                    pltpu.async_copy(href.at[c, s5].at[sidx_v.at[j]],
                                     rows_v.at[pl.ds(j * 128, 128)], sem)
                    for j in range(ec // 128)
                ]
                pltpu.sync_copy(eall.at[c, pg, s5, pl.ds(eb, ec)], m_v)
                for gcp in gcps:
                    gcp.wait()

                @plsc.parallel_loop(0, ec, unroll=8)
                def _cm(i):
                    a = m_v[i, pl.ds(0, 16)] + rows_v[i, pl.ds(0, 16)]
                    m_v[i, pl.ds(0, 16)] = jnp.maximum(a, 0.0)
                    b2 = m_v[i, pl.ds(16, 16)] + rows_v[i, pl.ds(16, 16)]
                    m_v[i, pl.ds(16, 16)] = jnp.maximum(b2, 0.0)

                for j in range(ec // 128):
                    pltpu.sync_copy(m_v.at[pl.ds(j * 128, 128)],
                                    agg_sp.at[didx_v.at[j]], add=True)
                return 0

            lax.fori_loop(0, nchunks, chunk, 0)
            plsc.subcore_barrier()
            pltpu.sync_copy(
                agg_sp.at[pl.ds(s * (N // NS), N // NS)],
                aref.at[c, s5, pl.ds(s * (N // NS), N // NS)])

    return msg(Hg, Hn, Eall, ei128)


def _readout(Hg, Hn, gid128):
    @functools.partial(
        pl.kernel, mesh=_mesh(), compiler_params=_SC_PARAMS,
        out_type=(jax.ShapeDtypeStruct((NC, G, B, HH), jnp.float32),
                  jax.ShapeDtypeStruct((NC, B, HH), jnp.float32)),
        scratch_types=[
            pltpu.VMEM((4, 128), jnp.int32),
            pltpu.VMEM((512, HH), jnp.float32),
            pltpu.VMEM((128, HH), jnp.float32),
            pltpu.VMEM((B // NS, HH), jnp.float32),
            pltpu.VMEM_SHARED((B, HH), jnp.float32),
            pltpu.VMEM_SHARED((B, HH), jnp.float32),
        ],
    )
    def rd(hg, hn, gidref, sums, cnt,
           didx_v, m_v, ones_v, z_v, sums_sp, cnt_sp):
        c = lax.axis_index("c")
        s = lax.axis_index("s")

        @plsc.parallel_loop(0, 128, unroll=8)
        def _io(i):
            ones_v[i, pl.ds(0, 16)] = jnp.ones((16,), jnp.float32)
            ones_v[i, pl.ds(16, 16)] = jnp.ones((16,), jnp.float32)

        @plsc.parallel_loop(0, B // NS, unroll=8)
        def _iz(i):
            z_v[i, pl.ds(0, 16)] = jnp.zeros((16,), jnp.float32)
            z_v[i, pl.ds(16, 16)] = jnp.zeros((16,), jnp.float32)

        pltpu.sync_copy(z_v, cnt_sp.at[pl.ds(s * (B // NS), B // NS)])
        plsc.subcore_barrier()

        def cchunk(ck, _):
            rb = s * 16 + ck * 4
            pltpu.sync_copy(gidref.at[c, pl.ds(rb, 4)], didx_v)
            for j in range(4):
                pltpu.sync_copy(ones_v, cnt_sp.at[didx_v.at[j]], add=True)
            return 0

        lax.fori_loop(0, 4, cchunk, 0)
        plsc.subcore_barrier()
        pltpu.sync_copy(cnt_sp.at[pl.ds(s * (B // NS), B // NS)],
                        cnt.at[c, pl.ds(s * (B // NS), B // NS)])

        for g in range(G):
            s5 = g % 5
            href = hg if g < 5 else hn
            gsel = 0 if s5 < R else 1
            pltpu.sync_copy(z_v, sums_sp.at[pl.ds(s * (B // NS), B // NS)])
            plsc.subcore_barrier()

            def schunk(ck, _):
                nb = s * (N // NS) + ck * 512
                rb = s * 16 + ck * 4
                pltpu.sync_copy(gidref.at[gsel, pl.ds(rb, 4)], didx_v)
                pltpu.sync_copy(href.at[c, s5, pl.ds(nb, 512)], m_v)
                for j in range(4):
                    pltpu.sync_copy(m_v.at[pl.ds(j * 128, 128)],
                                    sums_sp.at[didx_v.at[j]], add=True)
                return 0

            lax.fori_loop(0, 4, schunk, 0)
            plsc.subcore_barrier()
            pltpu.sync_copy(sums_sp.at[pl.ds(s * (B // NS), B // NS)],
                            sums.at[c, g, pl.ds(s * (B // NS), B // NS)])

    return rd(Hg, Hn, gid128)


def kernel(r_x, rn_x, p_x, pn_x, r_e, p_e, r_ei, p_ei, r_gid, p_gid,
           gin, ginn, Wq, Wk, Wp, bp):
    X4g = jnp.concatenate([r_x, p_x], axis=0).reshape(5, N // 4, 4 * D_IN)
    X4n = jnp.concatenate([rn_x, pn_x], axis=0).reshape(5, N // 4, 4 * D_E)
    EA4 = jnp.concatenate([r_e, p_e], axis=0).reshape(5, E // 4, 4 * D_E)
    ei128 = jnp.concatenate([r_ei, p_ei], axis=0).reshape(5, 2, E // 128, 128)
    gid128 = jnp.stack([r_gid, p_gid], axis=0).reshape(2, N // 128, 128)

    Hg4 = _node_proj(X4g, _expand_proj(gin['Wn']), _pack_bias(gin['bn']))
    Hn4 = _node_proj(X4n, _expand_proj(ginn['Wn']), _pack_bias(ginn['bn']))
    E4 = _edge_proj(EA4,
                    jnp.stack([_expand_proj(gin['We']),
                               _expand_proj(ginn['We'])], axis=0),
                    jnp.stack([_pack_bias(gin['be']),
                               _pack_bias(ginn['be'])], axis=0))
    Ev = E4.reshape(NC, 2, 5, E, HH)

    for l in range(DEPTH):
        Ag, An = _msg(Hg4.reshape(NC, 5, N, HH), Hn4.reshape(NC, 5, N, HH),
                      Ev, ei128)
        A1g, B1g = _expand_w1(gin['l%d_W1' % l])
        A1n, B1n = _expand_w1(ginn['l%d_W1' % l])
        Hg4 = _update(Hg4, Ag.reshape(NC, 5, N // 4, 128), A1g, B1g,
                      jnp.tile(gin['l%d_b1' % l], 4),
                      _expand_w2(gin['l%d_W2' % l]),
                      _pack_bias(gin['l%d_b2' % l]))
        Hn4 = _update(Hn4, An.reshape(NC, 5, N // 4, 128), A1n, B1n,
                      jnp.tile(ginn['l%d_b1' % l], 4),
                      _expand_w2(ginn['l%d_W2' % l]),
                      _pack_bias(ginn['l%d_b2' % l]))

    sums, cnt = _readout(Hg4.reshape(NC, 5, N, HH),
                         Hn4.reshape(NC, 5, N, HH), gid128)
    return _attn(sums, cnt, Wq, Wk, Wp, bp)


# speedup vs baseline: 7.3515x; 1.7196x over previous
import functools
import math

import jax
import jax.numpy as jnp
from jax import lax
from jax.experimental import pallas as pl
from jax.experimental.pallas import tpu as pltpu
from jax.experimental.pallas import tpu_sc as plsc

R, P, B = 3, 2, 512
N, E = 32768, 65536
D_IN, D_E, H, OUT = 155, 9, 64, 4
DEPTH = 3
NC, NS = 2, 16
HH = H // NC
G = 2 * (R + P)

_SC_PARAMS = pltpu.CompilerParams(use_tc_tiling_on_sc=False)


def _mesh():
    return plsc.VectorSubcoreMesh(core_axis_name="c", subcore_axis_name="s",
                                  num_cores=NC, num_subcores=NS)


def _expand_w1(W1):
    A = jnp.zeros((128, 256), W1.dtype)
    Bm = jnp.zeros((128, 256), W1.dtype)
    for j in range(4):
        A = A.at[32 * j:32 * j + 32, 64 * j:64 * j + 64].set(W1[:HH])
        Bm = Bm.at[32 * j:32 * j + 32, 64 * j:64 * j + 64].set(W1[HH:])
    return A, Bm


def _expand_w2(W2):
    C = jnp.zeros((256, 256), W2.dtype)
    for j in range(4):
        C = C.at[64 * j:64 * j + 64, 32 * j:32 * j + 32].set(W2[:, :HH])
        C = C.at[64 * j:64 * j + 64,
                 128 + 32 * j:128 + 32 * j + 32].set(W2[:, HH:])
    return C


def _expand_proj(W):
    D = W.shape[0]
    W4 = jnp.zeros((4 * D, 256), W.dtype)
    for j in range(4):
        W4 = W4.at[D * j:D * j + D, 32 * j:32 * j + 32].set(W[:, :HH])
        W4 = W4.at[D * j:D * j + D,
                   128 + 32 * j:128 + 32 * j + 32].set(W[:, HH:])
    return W4


def _pack_bias(b):
    return jnp.concatenate([jnp.tile(b[:HH], 4), jnp.tile(b[HH:], 4)])


def _dot(x, w):
    return jnp.dot(x, w, preferred_element_type=jnp.float32)


def _node_proj(X4, W4, b4):
    S, n4, D4 = X4.shape
    bn = 2048

    def body(x_ref, w_ref, b_ref, o_ref):
        y = _dot(x_ref[0], w_ref[...]) + b_ref[...]
        o_ref[0, 0] = y[:, :128]
        o_ref[1, 0] = y[:, 128:]

    return pl.pallas_call(
        body,
        grid=(S, n4 // bn),
        in_specs=[
            pl.BlockSpec((1, bn, D4), lambda g, i: (g, i, 0)),
            pl.BlockSpec((D4, 256), lambda g, i: (0, 0)),
            pl.BlockSpec((1, 256), lambda g, i: (0, 0)),
        ],
        out_specs=pl.BlockSpec((NC, 1, bn, 128), lambda g, i: (0, g, i, 0)),
        out_shape=jax.ShapeDtypeStruct((NC, S, n4, 128), jnp.float32),
    )(X4, W4, b4.reshape(1, 256))


def _edge_proj(EA4, W4s, b4s):
    be = 2048
    e4 = E // 4

    def body(e_ref, w_ref, b_ref, o_ref):
        y = _dot(e_ref[0], w_ref[0]) + b_ref[0]
        o_ref[0, 0, 0] = y[:, :128]
        o_ref[1, 0, 0] = y[:, 128:]

    return pl.pallas_call(
        body,
        grid=(2, 5, e4 // be),
        in_specs=[
            pl.BlockSpec((1, be, 4 * D_E), lambda p, g, i: (g, i, 0)),
            pl.BlockSpec((1, 4 * D_E, 256), lambda p, g, i: (p, 0, 0)),
            pl.BlockSpec((1, 1, 256), lambda p, g, i: (p, 0, 0)),
        ],
        out_specs=pl.BlockSpec((NC, 1, 1, be, 128),
                               lambda p, g, i: (0, p, g, i, 0)),
        out_shape=jax.ShapeDtypeStruct((NC, 2, 5, e4, 128), jnp.float32),
    )(EA4, W4s, b4s.reshape(2, 1, 256))


def _update(Hp4, Ap4, A1, B1, b1t, C2, b2p):
    bn = 2048
    n4 = N // 4

    def body(h_ref, a_ref, A_ref, B_ref, b1_ref, C_ref, b2_ref, o_ref):
        s0 = h_ref[0, 0] + a_ref[0, 0]
        s1 = h_ref[1, 0] + a_ref[1, 0]
        z1 = _dot(s0, A_ref[...]) + _dot(s1, B_ref[...]) + b1_ref[...]
        z1 = jnp.maximum(z1, 0.0)
        z2 = _dot(z1, C_ref[...]) + b2_ref[...]
        z2 = jnp.maximum(z2, 0.0)
        o_ref[0, 0] = z2[:, :128]
        o_ref[1, 0] = z2[:, 128:]

    return pl.pallas_call(
        body,
        grid=(5, n4 // bn),
        in_specs=[
            pl.BlockSpec((NC, 1, bn, 128), lambda g, i: (0, g, i, 0)),
            pl.BlockSpec((NC, 1, bn, 128), lambda g, i: (0, g, i, 0)),
            pl.BlockSpec((128, 256), lambda g, i: (0, 0)),
            pl.BlockSpec((128, 256), lambda g, i: (0, 0)),
            pl.BlockSpec((1, 256), lambda g, i: (0, 0)),
            pl.BlockSpec((256, 256), lambda g, i: (0, 0)),
            pl.BlockSpec((1, 256), lambda g, i: (0, 0)),
        ],
        out_specs=pl.BlockSpec((NC, 1, bn, 128), lambda g, i: (0, g, i, 0)),
        out_shape=jax.ShapeDtypeStruct((NC, 5, n4, 128), jnp.float32),
    )(Hp4, Ap4, A1, B1, b1t.reshape(1, 256), C2, b2p.reshape(1, 256))


def _attn(sums, cnt, Wq, Wk, Wp, bp):
    sc = 1.0 / math.sqrt(H)

    def body(s_ref, c_ref, wq_ref, wk_ref, wp_ref, bp_ref,
             o_out, o_ar, o_ap):
        feats = []
        for g in range(G):
            f = jnp.concatenate([s_ref[0, g], s_ref[1, g]], axis=-1)
            gsel = 0 if (g % 5) < R else 1
            cc = c_ref[gsel][:, :1]
            feats.append(f / jnp.maximum(cc, 1.0))
        r = feats[0:3]
        p = feats[3:5]
        rn = feats[5:8]
        pn = feats[8:10]
        r_rows = [r[0], r[1], r[2], r[0] + r[1], r[0] + r[2], r[1] + r[2]]
        rn_rows = [rn[0], rn[1], rn[2], rn[0] + rn[1], rn[0] + rn[2],
                   rn[1] + rn[2]]
        p_rows = [p[0], p[1], p[0] + p[1]]
        pn_rows = [pn[0], pn[1], pn[0] + pn[1]]

        def dotw(x, w):
            return jnp.dot(x, w, preferred_element_type=jnp.float32, precision=lax.Precision.HIGHEST)

        def att(q_list, k_list):
            acc = None
            for q in q_list:
                sij = jnp.concatenate(
                    [jnp.sum(q * k, axis=-1, keepdims=True) for k in k_list],
                    axis=-1) * sc
                m = jnp.max(sij, axis=-1, keepdims=True)
                e = jnp.exp(sij - m)
                a = e / jnp.sum(e, axis=-1, keepdims=True)
                acc = a if acc is None else acc + a
            return acc / float(len(q_list))

        qp = [dotw(x, wq_ref[...]) for x in p_rows]
        kr = [dotw(x, wk_ref[...]) for x in r_rows]
        att_reactant = att(qp, kr)
        qr = [dotw(x, wq_ref[...]) for x in r_rows]
        kp = [dotw(x, wk_ref[...]) for x in p_rows]
        att_product = att(qr, kp)

        reactant = sum(att_reactant[:, k:k + 1] * (r_rows[k] + rn_rows[k])
                       for k in range(6))
        product = sum(att_product[:, k:k + 1] * (p_rows[k] + pn_rows[k])
                      for k in range(3))
        reaction = reactant - product
        o_out[...] = jnp.dot(reaction, wp_ref[...],
                             preferred_element_type=jnp.float32, precision=lax.Precision.HIGHEST) + bp_ref[...]
        o_ar[...] = att_reactant
        o_ap[...] = att_product

    return pl.pallas_call(
        body,
        out_shape=(
            jax.ShapeDtypeStruct((B, OUT), jnp.float32),
            jax.ShapeDtypeStruct((B, 6), jnp.float32),
            jax.ShapeDtypeStruct((B, 3), jnp.float32),
        ),
    )(sums, cnt, Wq, Wk, Wp, bp.reshape(1, OUT))


def _msg(Hg, Hn, Eall, ei128):
    ec = 128
    nch = E // NS // ec
    nsl = 4
    nper = N // NS
    zr = 256

    @functools.partial(
        pl.kernel, mesh=_mesh(), compiler_params=_SC_PARAMS,
        out_type=(jax.ShapeDtypeStruct((NC, 5, N, HH), jnp.float32),
                  jax.ShapeDtypeStruct((NC, 5, N, HH), jnp.float32)),
        scratch_types=[
            pltpu.VMEM((nch, 128), jnp.int32),
            pltpu.VMEM((nch, 128), jnp.int32),
            pltpu.VMEM((nsl, ec, HH), jnp.float32),
            pltpu.VMEM((nsl, ec, HH), jnp.float32),
            pltpu.VMEM((zr, HH), jnp.float32),
            pltpu.VMEM_SHARED((N, HH), jnp.float32),
        ] + [pltpu.SemaphoreType.DMA] * (2 * nsl),
    )
    def msg(hg, hn, eall, ei128ref, agg_g, agg_n,
            sidx_v, didx_v, rows_v, m_v, z_v, agg_sp, *sems):
        c = lax.axis_index("c")
        s = lax.axis_index("s")
        sG, sS = sems[:nsl], sems[nsl:]

        @plsc.parallel_loop(0, zr, unroll=8)
        def _zz(i):
            z_v[i, pl.ds(0, 16)] = jnp.zeros((16,), jnp.float32)
            z_v[i, pl.ds(16, 16)] = jnp.zeros((16,), jnp.float32)

        for g in range(G):
            pg, s5 = (0, g) if g < 5 else (1, g - 5)
            href = hg if g < 5 else hn
            aref = agg_g if g < 5 else agg_n
            ebase = s * (E // NS)
            rb = s * nch
            for q in range(nper // zr):
                pltpu.sync_copy(z_v, agg_sp.at[pl.ds(s * nper + q * zr, zr)])
            pltpu.sync_copy(ei128ref.at[s5, 0, pl.ds(rb, nch)], sidx_v)
            pltpu.sync_copy(ei128ref.at[s5, 1, pl.ds(rb, nch)], didx_v)
            plsc.subcore_barrier()

            def issue_ge(ck, k):
                pltpu.async_copy(href.at[c, s5].at[sidx_v.at[ck]],
                                 rows_v.at[k], sG[k])
                pltpu.async_copy(
                    eall.at[c, pg, s5, pl.ds(ebase + ck * ec, ec)],
                    m_v.at[k], sG[k])

            def wait_ge(k):
                pltpu.make_async_copy(href.at[c, s5].at[sidx_v.at[0]],
                                      rows_v.at[k], sG[k]).wait()
                pltpu.make_async_copy(eall.at[c, pg, s5, pl.ds(ebase, ec)],
                                      m_v.at[k], sG[k]).wait()

            def issue_sc(ck, k):
                pltpu.async_copy(m_v.at[k], agg_sp.at[didx_v.at[ck]],
                                 sS[k], add=True)

            def wait_sc(k):
                pltpu.make_async_copy(m_v.at[k], agg_sp.at[didx_v.at[0]],
                                      sS[k]).wait()

            for k in range(nsl - 1):
                issue_ge(k, k)

            def pair_body(i, _):
                for k in range(nsl):
                    ck = nsl * i + k
                    wait_ge(k)
                    mk = m_v.at[k]
                    rk = rows_v.at[k]

                    @plsc.parallel_loop(0, ec, unroll=4)
                    def _cm(r):
                        a0 = mk[r, pl.ds(0, 16)] + rk[r, pl.ds(0, 16)]
                        mk[r, pl.ds(0, 16)] = jnp.maximum(a0, 0.0)
                        a1 = mk[r, pl.ds(16, 16)] + rk[r, pl.ds(16, 16)]
                        mk[r, pl.ds(16, 16)] = jnp.maximum(a1, 0.0)

                    issue_sc(ck, k)
                    kn = (k + nsl - 1) % nsl

                    @pl.when(ck + nsl - 1 < nch)
                    def _():
                        @pl.when(ck >= 1)
                        def _():
                            wait_sc(kn)
                        issue_ge(ck + nsl - 1, kn)
                return 0

            lax.fori_loop(0, nch // nsl, pair_body, 0)
            for k in range(nsl):
                wait_sc(k)
            plsc.subcore_barrier()
            pltpu.sync_copy(agg_sp.at[pl.ds(s * nper, nper)],
                            aref.at[c, s5, pl.ds(s * nper, nper)])

    return msg(Hg, Hn, Eall, ei128)


def _readout(Hg, Hn, gid128):
    @functools.partial(
        pl.kernel, mesh=_mesh(), compiler_params=_SC_PARAMS,
        out_type=(jax.ShapeDtypeStruct((NC, G, B, HH), jnp.float32),
                  jax.ShapeDtypeStruct((NC, B, HH), jnp.float32)),
        scratch_types=[
            pltpu.VMEM((4, 128), jnp.int32),
            pltpu.VMEM((512, HH), jnp.float32),
            pltpu.VMEM((128, HH), jnp.float32),
            pltpu.VMEM((B // NS, HH), jnp.float32),
            pltpu.VMEM_SHARED((B, HH), jnp.float32),
            pltpu.VMEM_SHARED((B, HH), jnp.float32),
        ],
    )
    def rd(hg, hn, gidref, sums, cnt,
           didx_v, m_v, ones_v, z_v, sums_sp, cnt_sp):
        c = lax.axis_index("c")
        s = lax.axis_index("s")

        @plsc.parallel_loop(0, 128, unroll=8)
        def _io(i):
            ones_v[i, pl.ds(0, 16)] = jnp.ones((16,), jnp.float32)
            ones_v[i, pl.ds(16, 16)] = jnp.ones((16,), jnp.float32)

        @plsc.parallel_loop(0, B // NS, unroll=8)
        def _iz(i):
            z_v[i, pl.ds(0, 16)] = jnp.zeros((16,), jnp.float32)
            z_v[i, pl.ds(16, 16)] = jnp.zeros((16,), jnp.float32)

        pltpu.sync_copy(z_v, cnt_sp.at[pl.ds(s * (B // NS), B // NS)])
        plsc.subcore_barrier()

        def cchunk(ck, _):
            rb = s * 16 + ck * 4
            pltpu.sync_copy(gidref.at[c, pl.ds(rb, 4)], didx_v)
            for j in range(4):
                pltpu.sync_copy(ones_v, cnt_sp.at[didx_v.at[j]], add=True)
            return 0

        lax.fori_loop(0, 4, cchunk, 0)
        plsc.subcore_barrier()
        pltpu.sync_copy(cnt_sp.at[pl.ds(s * (B // NS), B // NS)],
                        cnt.at[c, pl.ds(s * (B // NS), B // NS)])

        for g in range(G):
            s5 = g % 5
            href = hg if g < 5 else hn
            gsel = 0 if s5 < R else 1
            pltpu.sync_copy(z_v, sums_sp.at[pl.ds(s * (B // NS), B // NS)])
            plsc.subcore_barrier()

            def schunk(ck, _):
                nb = s * (N // NS) + ck * 512
                rb = s * 16 + ck * 4
                pltpu.sync_copy(gidref.at[gsel, pl.ds(rb, 4)], didx_v)
                pltpu.sync_copy(href.at[c, s5, pl.ds(nb, 512)], m_v)
                for j in range(4):
                    pltpu.sync_copy(m_v.at[pl.ds(j * 128, 128)],
                                    sums_sp.at[didx_v.at[j]], add=True)
                return 0

            lax.fori_loop(0, 4, schunk, 0)
            plsc.subcore_barrier()
            pltpu.sync_copy(sums_sp.at[pl.ds(s * (B // NS), B // NS)],
                            sums.at[c, g, pl.ds(s * (B // NS), B // NS)])

    return rd(Hg, Hn, gid128)


def kernel(r_x, rn_x, p_x, pn_x, r_e, p_e, r_ei, p_ei, r_gid, p_gid,
           gin, ginn, Wq, Wk, Wp, bp):
    X4g = jnp.concatenate([r_x, p_x], axis=0).reshape(5, N // 4, 4 * D_IN)
    X4n = jnp.concatenate([rn_x, pn_x], axis=0).reshape(5, N // 4, 4 * D_E)
    EA4 = jnp.concatenate([r_e, p_e], axis=0).reshape(5, E // 4, 4 * D_E)
    ei128 = jnp.concatenate([r_ei, p_ei], axis=0).reshape(5, 2, E // 128, 128)
    gid128 = jnp.stack([r_gid, p_gid], axis=0).reshape(2, N // 128, 128)

    Hg4 = _node_proj(X4g, _expand_proj(gin['Wn']), _pack_bias(gin['bn']))
    Hn4 = _node_proj(X4n, _expand_proj(ginn['Wn']), _pack_bias(ginn['bn']))
    E4 = _edge_proj(EA4,
                    jnp.stack([_expand_proj(gin['We']),
                               _expand_proj(ginn['We'])], axis=0),
                    jnp.stack([_pack_bias(gin['be']),
                               _pack_bias(ginn['be'])], axis=0))
    Ev = E4.reshape(NC, 2, 5, E, HH)

    for l in range(DEPTH):
        Ag, An = _msg(Hg4.reshape(NC, 5, N, HH), Hn4.reshape(NC, 5, N, HH),
                      Ev, ei128)
        A1g, B1g = _expand_w1(gin['l%d_W1' % l])
        A1n, B1n = _expand_w1(ginn['l%d_W1' % l])
        Hg4 = _update(Hg4, Ag.reshape(NC, 5, N // 4, 128), A1g, B1g,
                      jnp.tile(gin['l%d_b1' % l], 4),
                      _expand_w2(gin['l%d_W2' % l]),
                      _pack_bias(gin['l%d_b2' % l]))
        Hn4 = _update(Hn4, An.reshape(NC, 5, N // 4, 128), A1n, B1n,
                      jnp.tile(ginn['l%d_b1' % l], 4),
                      _expand_w2(ginn['l%d_W2' % l]),
                      _pack_bias(ginn['l%d_b2' % l]))

    sums, cnt = _readout(Hg4.reshape(NC, 5, N, HH),
                         Hn4.reshape(NC, 5, N, HH), gid128)
    return _attn(sums, cnt, Wq, Wk, Wp, bp)
